# baseline (speedup 1.0000x reference)
import functools

import jax
import jax.numpy as jnp
from jax import lax
from jax.experimental import pallas as pl
from jax.experimental.pallas import tpu as pltpu
from jax.experimental.pallas import tpu_sc as plsc

_HIDDEN = 128
_BCONST = 3.2

_MESH = plsc.ScalarSubcoreMesh(axis_name="c", num_cores=1)


@functools.partial(
    pl.kernel,
    out_type=jax.ShapeDtypeStruct((1, 1), jnp.float32),
    mesh=_MESH,
    scratch_types=[
        pltpu.SMEM((1, 2), jnp.int32),
        pltpu.SMEM((1, _HIDDEN), jnp.float32),
        pltpu.SMEM((1, _HIDDEN), jnp.float32),
        pltpu.SMEM((1, 8), jnp.float32),
        pltpu.SMEM((1, 8), jnp.float32),
        pltpu.SMEM((1, 1), jnp.float32),
        pltpu.SemaphoreType.DMA,
    ],
    compiler_params=pltpu.CompilerParams(needs_layout_passes=False),
)
def _autorec_scs(row, P, Q, b_u, b_i, out, row_s, p_s, q_s, bu_s, bi_s, out_s, sem):
    pltpu.sync_copy(row, row_s)
    u = row_s[0, 0]
    i = row_s[0, 1]

    cp = pltpu.async_copy(P.at[pl.ds(u, 1)], p_s, sem)
    cq = pltpu.async_copy(Q.at[pl.ds(i, 1)], q_s, sem)
    cbu = pltpu.async_copy(b_u.at[pl.ds(u // 8, 1)], bu_s, sem)
    cbi = pltpu.async_copy(b_i.at[pl.ds(i // 8, 1)], bi_s, sem)
    cp.wait()
    cq.wait()
    cbu.wait()
    cbi.wait()

    def body(j, acc):
        return acc + p_s[0, j] * q_s[0, j]

    dot = lax.fori_loop(0, _HIDDEN, body, jnp.float32(0.0))
    out_s[0, 0] = dot + bu_s[0, u % 8] + bi_s[0, i % 8] + jnp.float32(_BCONST)
    pltpu.sync_copy(out_s, out)


def kernel(row, P, Q, b_u, b_i):
    out = _autorec_scs(
        row.reshape(1, 2),
        P,
        Q,
        b_u.reshape(-1, 8),
        b_i.reshape(-1, 8),
    )
    return jnp.reshape(out, ())


# speedup vs baseline: 1.3855x; 1.3855x over previous
import functools

import jax
import jax.numpy as jnp
from jax import lax
from jax.experimental import pallas as pl
from jax.experimental.pallas import tpu as pltpu
from jax.experimental.pallas import tpu_sc as plsc

_HIDDEN = 128
_LANES = 16
_BCONST = 3.2

_MESH = plsc.VectorSubcoreMesh(
    core_axis_name="c", subcore_axis_name="s", num_cores=1, num_subcores=1
)


@functools.partial(
    pl.kernel,
    out_type=jax.ShapeDtypeStruct((1,), jnp.float32),
    mesh=_MESH,
    scratch_types=[
        pltpu.VMEM((_LANES,), jnp.int32),
        pltpu.VMEM((_LANES, _HIDDEN), jnp.float32),
        pltpu.VMEM((_LANES, _HIDDEN), jnp.float32),
        pltpu.VMEM((_LANES,), jnp.float32),
        pltpu.VMEM((_LANES,), jnp.float32),
        pltpu.VMEM((_LANES,), jnp.float32),
        pltpu.SemaphoreType.DMA,
    ],
    compiler_params=pltpu.CompilerParams(
        needs_layout_passes=False,
        skip_device_barrier=True,
        disable_bounds_checks=True,
        disable_semaphore_checks=True,
    ),
)
def _autorec_sc(row, P, Q, b_u, b_i, out, idx_v, p_v, q_v, bu_v, bi_v, out_v, sem):
    wid = lax.axis_index("s") + lax.axis_index("c")

    @pl.when(wid == 0)
    def _():
        idx_v[...] = jnp.zeros((_LANES,), jnp.int32)
        pltpu.sync_copy(row, idx_v.at[pl.ds(0, 2)])

        cp = pltpu.async_copy(P.at[idx_v], p_v, sem)
        cq = pltpu.async_copy(Q.at[idx_v], q_v, sem)
        cbu = pltpu.async_copy(b_u.at[idx_v], bu_v, sem)
        cbi = pltpu.async_copy(b_i.at[idx_v], bi_v, sem)
        cp.wait()
        cq.wait()
        cbu.wait()
        cbi.wait()

        lane = lax.iota(jnp.int32, _LANES)
        zero = jnp.zeros((_LANES,), jnp.float32)
        acc = jnp.where(lane == 0, bu_v[...], zero)
        acc = acc + jnp.where(lane == 1, bi_v[...], zero)

        def chunk(j, a):
            off = pl.multiple_of(j * _LANES, _LANES)
            return a + p_v[0, pl.ds(off, _LANES)] * q_v[1, pl.ds(off, _LANES)]

        acc = lax.fori_loop(0, _HIDDEN // _LANES, chunk, acc)

        total = jnp.sum(acc) + jnp.float32(_BCONST)
        out_v[...] = jnp.full((_LANES,), total, jnp.float32)
        pltpu.sync_copy(out_v.at[pl.ds(0, 1)], out)


def kernel(row, P, Q, b_u, b_i):
    return jnp.reshape(_autorec_sc(row, P, Q, b_u, b_i), ())


# speedup vs baseline: 1.4630x; 1.0560x over previous
import functools

import jax
import jax.numpy as jnp
from jax import lax
from jax.experimental import pallas as pl
from jax.experimental.pallas import tpu as pltpu
from jax.experimental.pallas import tpu_sc as plsc

_HIDDEN = 128
_LANES = 16
_BCONST = 3.2

_MESH = plsc.VectorSubcoreMesh(
    core_axis_name="c", subcore_axis_name="s", num_cores=1, num_subcores=1
)


@functools.partial(
    pl.kernel,
    out_type=jax.ShapeDtypeStruct((1,), jnp.float32),
    mesh=_MESH,
    scratch_types=[
        pltpu.VMEM((_LANES,), jnp.int32),
        pltpu.VMEM((2, _HIDDEN), jnp.float32),
        pltpu.VMEM((2, _HIDDEN), jnp.float32),
        pltpu.VMEM((_LANES,), jnp.float32),
        pltpu.VMEM((_LANES,), jnp.float32),
        pltpu.VMEM((_LANES,), jnp.float32),
        pltpu.SemaphoreType.DMA,
    ],
    compiler_params=pltpu.CompilerParams(
        needs_layout_passes=False,
        skip_device_barrier=True,
        disable_bounds_checks=True,
        disable_semaphore_checks=True,
    ),
)
def _autorec_sc(row, P, Q, b_u, b_i, out, idx_v, p_v, q_v, bu_v, bi_v, out_v, sem):
    wid = lax.axis_index("s") + lax.axis_index("c")

    @pl.when(wid == 0)
    def _():
        idx_v[...] = jnp.zeros((_LANES,), jnp.int32)
        pltpu.sync_copy(row, idx_v.at[pl.ds(0, 2)])

        cp = pltpu.async_copy(P.at[idx_v.at[pl.ds(0, 2)]], p_v, sem)
        cq = pltpu.async_copy(Q.at[idx_v.at[pl.ds(0, 2)]], q_v, sem)
        cbu = pltpu.async_copy(b_u.at[idx_v], bu_v, sem)
        cbi = pltpu.async_copy(b_i.at[idx_v], bi_v, sem)
        cp.wait()
        cq.wait()

        acc = p_v[0, pl.ds(0, _LANES)] * q_v[1, pl.ds(0, _LANES)]
        for j in range(1, _HIDDEN // _LANES):
            acc = acc + p_v[0, pl.ds(j * _LANES, _LANES)] * q_v[1, pl.ds(j * _LANES, _LANES)]

        cbu.wait()
        cbi.wait()
        lane = lax.iota(jnp.int32, _LANES)
        zero = jnp.zeros((_LANES,), jnp.float32)
        acc = acc + jnp.where(lane == 0, bu_v[...], zero)
        acc = acc + jnp.where(lane == 1, bi_v[...], zero)

        total = jnp.sum(acc) + jnp.float32(_BCONST)
        out_v[...] = jnp.full((_LANES,), total, jnp.float32)
        pltpu.sync_copy(out_v.at[pl.ds(0, 1)], out)


def kernel(row, P, Q, b_u, b_i):
    return jnp.reshape(_autorec_sc(row, P, Q, b_u, b_i), ())
